# baseline (device time: 8410 ns/iter reference)
import jax
import jax.numpy as jnp
from jax import lax
from jax.experimental import pallas as pl
from jax.experimental.pallas import tpu as pltpu

N_DEV = 4


def kernel(x):
    m, n = x.shape

    def body(
        x_hbm, out_hbm, x_vmem, out_vmem, total_ref, comm_ref,
        copy_sems, send_sems, recv_sems,
    ):
        my_pos = lax.axis_index("i")

        barrier_sem = pltpu.get_barrier_semaphore()
        for off in (1, 2, 3):
            pl.semaphore_signal(
                barrier_sem, inc=1,
                device_id=((my_pos + off) % N_DEV,),
                device_id_type=pl.DeviceIdType.MESH,
            )

        copy_in = pltpu.make_async_copy(x_hbm, x_vmem, copy_sems.at[0])
        copy_in.start()
        copy_in.wait()

        t = x_vmem[:, :]
        size = m
        while size > 1:
            half = size // 2
            t = t[:half, :] * t[half:size, :]
            size = half
        total_ref[:, :] = t

        pl.semaphore_wait(barrier_sem, N_DEV - 1)

        sends = []
        for off in (1, 2, 3):
            tgt = (my_pos + off) % N_DEV
            rdma = pltpu.make_async_remote_copy(
                src_ref=total_ref,
                dst_ref=comm_ref.at[my_pos],
                send_sem=send_sems.at[off - 1],
                recv_sem=recv_sems.at[my_pos],
                device_id=(tgt,),
                device_id_type=pl.DeviceIdType.MESH,
            )
            rdma.start()
            sends.append(rdma)

        B = 32
        rows = m // B
        y3 = x_vmem[:, :].reshape(B, rows, n)
        k = 1
        while k < rows:
            shifted = jnp.concatenate(
                [jnp.ones((B, k, n), jnp.float32), y3[:, :-k, :]], axis=1
            )
            y3 = y3 * shifted
            k *= 2
        ebt = jnp.concatenate(
            [jnp.ones((1, 1, n), jnp.float32), y3[:-1, rows - 1 :, :]], axis=0
        )
        k = 1
        while k < B:
            shifted = jnp.concatenate(
                [jnp.ones((k, 1, n), jnp.float32), ebt[:-k, :, :]], axis=0
            )
            ebt = ebt * shifted
            k *= 2

        for j in range(N_DEV):
            @pl.when(j != my_pos)
            def _(j=j):
                recv = pltpu.make_async_remote_copy(
                    src_ref=total_ref,
                    dst_ref=comm_ref.at[j],
                    send_sem=send_sems.at[0],
                    recv_sem=recv_sems.at[j],
                    device_id=(j,),
                    device_id_type=pl.DeviceIdType.MESH,
                )
                recv.wait_recv()

        gathered = comm_ref[:, :, :]
        idx = lax.broadcasted_iota(jnp.int32, (N_DEV, 1, n), 0)
        factors = jnp.where(idx < my_pos, gathered, 1.0)
        prefix = factors[0] * factors[1] * factors[2] * factors[3]

        out_vmem[:, :] = (y3 * (ebt * prefix[None, :, :])).reshape(m, n)
        copy_out = pltpu.make_async_copy(out_vmem, out_hbm, copy_sems.at[1])
        copy_out.start()
        copy_out.wait()

        for s in sends:
            s.wait_send()

    return pl.pallas_call(
        body,
        out_shape=jax.ShapeDtypeStruct((m, n), jnp.float32),
        in_specs=[pl.BlockSpec(memory_space=pl.ANY)],
        out_specs=pl.BlockSpec(memory_space=pl.ANY),
        scratch_shapes=[
            pltpu.VMEM((m, n), jnp.float32),
            pltpu.VMEM((m, n), jnp.float32),
            pltpu.VMEM((1, n), jnp.float32),
            pltpu.VMEM((N_DEV, 1, n), jnp.float32),
            pltpu.SemaphoreType.DMA((2,)),
            pltpu.SemaphoreType.DMA((3,)),
            pltpu.SemaphoreType.DMA((N_DEV,)),
        ],
        compiler_params=pltpu.CompilerParams(collective_id=0),
    )(x)
